# baseline (speedup 1.0000x reference)
import functools

import jax
import jax.numpy as jnp
from jax import lax
from jax.experimental import pallas as pl
from jax.experimental.pallas import tpu as pltpu
from jax.experimental.pallas import tpu_sc as plsc

_N = 10000
_S = 32
_D = 128
_U = 128

_B = 400
_S_SC = 3200
_R = _N - _S_SC

_NC = 2
_NS = 16
_NW = _NC * _NS
_C = 4
_PER_W = _S_SC // _NW
_CHUNKS = _PER_W // _C
_INV = 1.0 / (_S + 1)


@functools.partial(
    pl.kernel,
    out_type=jax.ShapeDtypeStruct((_S_SC, _D), jnp.float32),
    mesh=plsc.VectorSubcoreMesh(core_axis_name="c", subcore_axis_name="s"),
    scratch_types=[
        pltpu.VMEM((_C, _S, _D), jnp.float32),
        pltpu.VMEM((_C, _D), jnp.float32),
        pltpu.VMEM((_C, _D), jnp.float32),
    ],
)
def _sc_aggregate(x_hbm, nb_hbm, h_hbm, nb_v, x_v, h_v):
    w = lax.axis_index("s") * _NC + lax.axis_index("c")
    row_w = w * _PER_W

    def chunk(t, carry):
        base = row_w + t * _C
        pltpu.sync_copy(nb_hbm.at[pl.ds(_R + base, _C)], nb_v)
        pltpu.sync_copy(x_hbm.at[pl.ds(_R + base, _C)], x_v)
        for i in range(_C):
            for j in range(_D // 16):
                col = pl.ds(j * 16, 16)
                acc = x_v[i, col]
                for s in range(_S):
                    acc = acc + nb_v[i, s, col]
                h_v[i, col] = acc * _INV
        pltpu.sync_copy(h_v, h_hbm.at[pl.ds(base, _C)])
        return carry

    lax.fori_loop(0, _CHUNKS, chunk, 0)


def _fused_body(x_ref, nb_ref, w_ref, b_ref, o_ref):
    s = jnp.sum(nb_ref[...], axis=1) + x_ref[...]
    h = s * _INV
    out = jnp.dot(h, w_ref[...], preferred_element_type=jnp.float32)
    o_ref[...] = jnp.maximum(out + b_ref[...], 0.0)


def _tc_fused(x, neigh_x, w, bias2):
    grid = _R // _B
    return pl.pallas_call(
        _fused_body,
        grid=(grid,),
        in_specs=[
            pl.BlockSpec((_B, _D), lambda i: (i, 0)),
            pl.BlockSpec((_B, _S, _D), lambda i: (i, 0, 0)),
            pl.BlockSpec((_D, _U), lambda i: (0, 0)),
            pl.BlockSpec((1, _U), lambda i: (0, 0)),
        ],
        out_specs=pl.BlockSpec((_B, _U), lambda i: (i, 0)),
        out_shape=jax.ShapeDtypeStruct((_R, _U), jnp.float32),
        compiler_params=pltpu.CompilerParams(
            dimension_semantics=("arbitrary",),
        ),
    )(x, neigh_x, w, bias2)


def _proj_body(h_ref, w_ref, b_ref, o_ref):
    out = jnp.dot(h_ref[...], w_ref[...], preferred_element_type=jnp.float32)
    o_ref[...] = jnp.maximum(out + b_ref[...], 0.0)


def _tc_project(h, w, bias2):
    grid = _S_SC // _B
    return pl.pallas_call(
        _proj_body,
        grid=(grid,),
        in_specs=[
            pl.BlockSpec((_B, _D), lambda i: (i, 0)),
            pl.BlockSpec((_D, _U), lambda i: (0, 0)),
            pl.BlockSpec((1, _U), lambda i: (0, 0)),
        ],
        out_specs=pl.BlockSpec((_B, _U), lambda i: (i, 0)),
        out_shape=jax.ShapeDtypeStruct((_S_SC, _U), jnp.float32),
        compiler_params=pltpu.CompilerParams(
            dimension_semantics=("arbitrary",),
        ),
    )(h, w, bias2)


def kernel(x, neigh_x, kernel, bias):
    bias2 = bias.reshape(1, _U)
    h_sc = _sc_aggregate(x, neigh_x)
    out_head = _tc_fused(x, neigh_x, kernel, bias2)
    out_tail = _tc_project(h_sc, kernel, bias2)
    return jnp.concatenate([out_head, out_tail], axis=0)


# speedup vs baseline: 1.3967x; 1.3967x over previous
import functools

import jax
import jax.numpy as jnp
from jax import lax
from jax.experimental import pallas as pl
from jax.experimental.pallas import tpu as pltpu
from jax.experimental.pallas import tpu_sc as plsc

_N = 10000
_S = 32
_D = 128
_U = 128

_B = 496
_S_SC = 2560
_R = _N - _S_SC

_BP = 512
_NC = 2
_NS = 16
_NW = _NC * _NS
_C = 8
_NBUF = 2
_PER_W = _S_SC // _NW
_CHUNKS = _PER_W // _C
_INV = 1.0 / (_S + 1)


@functools.partial(
    pl.kernel,
    out_type=jax.ShapeDtypeStruct((_S_SC, _D), jnp.float32),
    mesh=plsc.VectorSubcoreMesh(core_axis_name="c", subcore_axis_name="s"),
    scratch_types=[
        pltpu.VMEM((_NBUF, _C, _S, _D), jnp.float32),
        pltpu.VMEM((_NBUF, _C, _D), jnp.float32),
        pltpu.SemaphoreType.DMA,
        pltpu.SemaphoreType.DMA,
        pltpu.SemaphoreType.DMA,
        pltpu.SemaphoreType.DMA,
    ],
)
def _sc_aggregate(x_hbm, nb_hbm, h_hbm, nb_v, h_v, sem_nb0, sem_nb1,
                  sem_h0, sem_h1):
    del x_hbm
    w = lax.axis_index("s") * _NC + lax.axis_index("c")
    row_w = w * _PER_W
    sems_nb = (sem_nb0, sem_nb1)
    sems_h = (sem_h0, sem_h1)

    def nb_copy(t, b):
        src = nb_hbm.at[pl.ds(_R + row_w + t * _C, _C)]
        return pltpu.make_async_copy(src, nb_v.at[b], sems_nb[b])

    def h_copy(t, b):
        dst = h_hbm.at[pl.ds(row_w + t * _C, _C)]
        return pltpu.make_async_copy(h_v.at[b], dst, sems_h[b])

    for b in range(_NBUF):
        nb_copy(b, b).start()

    def loop_body(g, carry):
        for b in range(_NBUF):
            t = g + b
            nb_copy(t, b).wait()

            @pl.when(t >= _NBUF)
            def _():
                h_copy(t - _NBUF, b).wait()

            for i in range(_C):
                for j in range(_D // 16):
                    col = pl.ds(j * 16, 16)
                    acc = nb_v[b, i, 0, col]
                    for s in range(1, _S):
                        acc = acc + nb_v[b, i, s, col]
                    h_v[b, i, col] = acc

            h_copy(t, b).start()

            @pl.when(t + _NBUF < _CHUNKS)
            def _():
                nb_copy(t + _NBUF, b).start()
        return carry

    lax.fori_loop(0, _CHUNKS // _NBUF, lambda g, c: loop_body(g * _NBUF, c), 0)

    for b in range(_NBUF):
        h_copy(_CHUNKS - _NBUF + b, b).wait()


def _fused_body(x_ref, nb_ref, w_ref, b_ref, o_ref):
    s = jnp.sum(nb_ref[...], axis=1) + x_ref[...]
    h = s * _INV
    out = jnp.dot(h, w_ref[...], preferred_element_type=jnp.float32)
    o_ref[...] = jnp.maximum(out + b_ref[...], 0.0)


def _tc_fused(x, neigh_x, w, bias2):
    grid = _R // _B
    return pl.pallas_call(
        _fused_body,
        grid=(grid,),
        in_specs=[
            pl.BlockSpec((_B, _D), lambda i: (i, 0)),
            pl.BlockSpec((_B, _S, _D), lambda i: (i, 0, 0)),
            pl.BlockSpec((_D, _U), lambda i: (0, 0)),
            pl.BlockSpec((1, _U), lambda i: (0, 0)),
        ],
        out_specs=pl.BlockSpec((_B, _U), lambda i: (i, 0)),
        out_shape=jax.ShapeDtypeStruct((_R, _U), jnp.float32),
        compiler_params=pltpu.CompilerParams(
            dimension_semantics=("arbitrary",),
        ),
    )(x, neigh_x, w, bias2)


def _proj_body(h_ref, x_ref, w_ref, b_ref, o_ref):
    h = (h_ref[...] + x_ref[...]) * _INV
    out = jnp.dot(h, w_ref[...], preferred_element_type=jnp.float32)
    o_ref[...] = jnp.maximum(out + b_ref[...], 0.0)


def _tc_project(h, x_tail, w, bias2):
    grid = _S_SC // _BP
    return pl.pallas_call(
        _proj_body,
        grid=(grid,),
        in_specs=[
            pl.BlockSpec((_BP, _D), lambda i: (i, 0)),
            pl.BlockSpec((_BP, _D), lambda i: (i, 0)),
            pl.BlockSpec((_D, _U), lambda i: (0, 0)),
            pl.BlockSpec((1, _U), lambda i: (0, 0)),
        ],
        out_specs=pl.BlockSpec((_BP, _U), lambda i: (i, 0)),
        out_shape=jax.ShapeDtypeStruct((_S_SC, _U), jnp.float32),
        compiler_params=pltpu.CompilerParams(
            dimension_semantics=("arbitrary",),
        ),
    )(h, x_tail, w, bias2)


def kernel(x, neigh_x, kernel, bias):
    bias2 = bias.reshape(1, _U)
    h_sc = _sc_aggregate(x, neigh_x)
    out_head = _tc_fused(x, neigh_x, kernel, bias2)
    out_tail = _tc_project(h_sc, lax.slice(x, (_R, 0), (_N, _D)), kernel,
                           bias2)
    return jnp.concatenate([out_head, out_tail], axis=0)


# speedup vs baseline: 1.8116x; 1.2971x over previous
import functools

import jax
import jax.numpy as jnp
from jax import lax
from jax.experimental import pallas as pl
from jax.experimental.pallas import tpu as pltpu
from jax.experimental.pallas import tpu_sc as plsc

_N = 10000
_S = 32
_D = 128
_U = 128

_B = 496
_S_SC = 2560
_R = _N - _S_SC

_BP = 512
_NC = 2
_NS = 16
_NW = _NC * _NS
_C = 8
_NBUF = 2
_PER_W = _S_SC // _NW
_CHUNKS = _PER_W // _C
_INV = 1.0 / (_S + 1)


@functools.partial(
    pl.kernel,
    out_type=jax.ShapeDtypeStruct((_S_SC, _D), jnp.float32),
    mesh=plsc.VectorSubcoreMesh(core_axis_name="c", subcore_axis_name="s"),
    scratch_types=[
        pltpu.VMEM((_NBUF, _C, _S, _D), jnp.float32),
        pltpu.VMEM((_NBUF, _C, _D), jnp.float32),
        pltpu.SemaphoreType.DMA,
        pltpu.SemaphoreType.DMA,
        pltpu.SemaphoreType.DMA,
        pltpu.SemaphoreType.DMA,
    ],
)
def _sc_aggregate(x_hbm, nb_hbm, h_hbm, nb_v, h_v, sem_nb0, sem_nb1,
                  sem_h0, sem_h1):
    del x_hbm
    w = lax.axis_index("s") * _NC + lax.axis_index("c")
    row_w = w * _PER_W
    sems_nb = (sem_nb0, sem_nb1)
    sems_h = (sem_h0, sem_h1)

    def nb_copy(t, b):
        src = nb_hbm.at[pl.ds(_R + row_w + t * _C, _C)]
        return pltpu.make_async_copy(src, nb_v.at[b], sems_nb[b])

    def h_copy(t, b):
        dst = h_hbm.at[pl.ds(row_w + t * _C, _C)]
        return pltpu.make_async_copy(h_v.at[b], dst, sems_h[b])

    for b in range(_NBUF):
        nb_copy(b, b).start()

    def loop_body(g, carry):
        for b in range(_NBUF):
            t = g + b
            nb_copy(t, b).wait()

            @pl.when(t >= _NBUF)
            def _():
                h_copy(t - _NBUF, b).wait()

            def node(i, c):
                for j in range(_D // 16):
                    col = pl.ds(j * 16, 16)
                    acc = nb_v[b, i, 0, col]
                    for s in range(1, _S):
                        acc = acc + nb_v[b, i, s, col]
                    h_v[b, i, col] = acc
                return c

            lax.fori_loop(0, _C, node, 0)

            h_copy(t, b).start()

            @pl.when(t + _NBUF < _CHUNKS)
            def _():
                nb_copy(t + _NBUF, b).start()
        return carry

    lax.fori_loop(0, _CHUNKS // _NBUF, lambda g, c: loop_body(g * _NBUF, c), 0)

    for b in range(_NBUF):
        h_copy(_CHUNKS - _NBUF + b, b).wait()


def _fused_body(x_ref, nb_ref, w_ref, b_ref, o_ref):
    s = jnp.sum(nb_ref[...], axis=1) + x_ref[...]
    h = s * _INV
    out = jnp.dot(h, w_ref[...], preferred_element_type=jnp.float32)
    o_ref[...] = jnp.maximum(out + b_ref[...], 0.0)


def _tc_fused(x, neigh_x, w, bias2):
    grid = _R // _B
    return pl.pallas_call(
        _fused_body,
        grid=(grid,),
        in_specs=[
            pl.BlockSpec((_B, _D), lambda i: (i, 0)),
            pl.BlockSpec((_B, _S, _D), lambda i: (i, 0, 0)),
            pl.BlockSpec((_D, _U), lambda i: (0, 0)),
            pl.BlockSpec((1, _U), lambda i: (0, 0)),
        ],
        out_specs=pl.BlockSpec((_B, _U), lambda i: (i, 0)),
        out_shape=jax.ShapeDtypeStruct((_R, _U), jnp.float32),
        compiler_params=pltpu.CompilerParams(
            dimension_semantics=("arbitrary",),
        ),
    )(x, neigh_x, w, bias2)


def _proj_body(h_ref, x_ref, w_ref, b_ref, o_ref):
    h = (h_ref[...] + x_ref[...]) * _INV
    out = jnp.dot(h, w_ref[...], preferred_element_type=jnp.float32)
    o_ref[...] = jnp.maximum(out + b_ref[...], 0.0)


def _tc_project(h, x_tail, w, bias2):
    grid = _S_SC // _BP
    return pl.pallas_call(
        _proj_body,
        grid=(grid,),
        in_specs=[
            pl.BlockSpec((_BP, _D), lambda i: (i, 0)),
            pl.BlockSpec((_BP, _D), lambda i: (i, 0)),
            pl.BlockSpec((_D, _U), lambda i: (0, 0)),
            pl.BlockSpec((1, _U), lambda i: (0, 0)),
        ],
        out_specs=pl.BlockSpec((_BP, _U), lambda i: (i, 0)),
        out_shape=jax.ShapeDtypeStruct((_S_SC, _U), jnp.float32),
        compiler_params=pltpu.CompilerParams(
            dimension_semantics=("arbitrary",),
        ),
    )(h, x_tail, w, bias2)


def kernel(x, neigh_x, kernel, bias):
    bias2 = bias.reshape(1, _U)
    h_sc = _sc_aggregate(x, neigh_x)
    out_head = _tc_fused(x, neigh_x, kernel, bias2)
    out_tail = _tc_project(h_sc, lax.slice(x, (_R, 0), (_N, _D)), kernel,
                           bias2)
    return jnp.concatenate([out_head, out_tail], axis=0)


# speedup vs baseline: 1.8170x; 1.0030x over previous
import functools

import jax
import jax.numpy as jnp
from jax import lax
from jax.experimental import pallas as pl
from jax.experimental.pallas import tpu as pltpu
from jax.experimental.pallas import tpu_sc as plsc

_N = 10000
_S = 32
_D = 128
_U = 128

_B = 496
_S_SC = 2560
_R = _N - _S_SC

_BP = 512
_NC = 2
_NS = 16
_NW = _NC * _NS
_C = 8
_NBUF = 2
_PER_W = _S_SC // _NW
_CHUNKS = _PER_W // _C
_INV = 1.0 / (_S + 1)


@functools.partial(
    pl.kernel,
    out_type=jax.ShapeDtypeStruct((_S_SC, _D), jnp.float32),
    mesh=plsc.VectorSubcoreMesh(core_axis_name="c", subcore_axis_name="s"),
    scratch_types=[
        pltpu.VMEM((_NBUF, _C, _S, _D), jnp.float32),
        pltpu.VMEM((_NBUF, _C, _D), jnp.float32),
        pltpu.SemaphoreType.DMA,
        pltpu.SemaphoreType.DMA,
        pltpu.SemaphoreType.DMA,
        pltpu.SemaphoreType.DMA,
        pltpu.SemaphoreType.DMA,
        pltpu.SemaphoreType.DMA,
    ],
)
def _sc_aggregate(x_hbm, nb_hbm, h_hbm, nb_v, h_v, sem_nb0, sem_nb1,
                  sem_nb2, sem_nb3, sem_h0, sem_h1):
    del x_hbm
    w = lax.axis_index("s") * _NC + lax.axis_index("c")
    row_w = w * _PER_W
    sems_nb = ((sem_nb0, sem_nb1), (sem_nb2, sem_nb3))
    sems_h = (sem_h0, sem_h1)
    _H = _C // 2

    def nb_copies(t, b):
        r0 = _R + row_w + t * _C
        return (
            pltpu.make_async_copy(
                nb_hbm.at[pl.ds(r0, _H)],
                nb_v.at[b, pl.ds(0, _H)], sems_nb[b][0]),
            pltpu.make_async_copy(
                nb_hbm.at[pl.ds(r0 + _H, _H)],
                nb_v.at[b, pl.ds(_H, _H)], sems_nb[b][1]),
        )

    def h_copy(t, b):
        dst = h_hbm.at[pl.ds(row_w + t * _C, _C)]
        return pltpu.make_async_copy(h_v.at[b], dst, sems_h[b])

    for b in range(_NBUF):
        for cp in nb_copies(b, b):
            cp.start()

    def loop_body(g, carry):
        for b in range(_NBUF):
            t = g + b
            for cp in nb_copies(t, b):
                cp.wait()

            @pl.when(t >= _NBUF)
            def _():
                h_copy(t - _NBUF, b).wait()

            def node(i, c):
                for j in range(_D // 16):
                    col = pl.ds(j * 16, 16)
                    acc = nb_v[b, i, 0, col]
                    for s in range(1, _S):
                        acc = acc + nb_v[b, i, s, col]
                    h_v[b, i, col] = acc
                return c

            lax.fori_loop(0, _C, node, 0)

            h_copy(t, b).start()

            @pl.when(t + _NBUF < _CHUNKS)
            def _():
                for cp in nb_copies(t + _NBUF, b):
                    cp.start()
        return carry

    lax.fori_loop(0, _CHUNKS // _NBUF, lambda g, c: loop_body(g * _NBUF, c), 0)

    for b in range(_NBUF):
        h_copy(_CHUNKS - _NBUF + b, b).wait()


def _fused_body(x_ref, nb_ref, w_ref, b_ref, o_ref):
    s = jnp.sum(nb_ref[...], axis=1) + x_ref[...]
    h = s * _INV
    out = jnp.dot(h, w_ref[...], preferred_element_type=jnp.float32)
    o_ref[...] = jnp.maximum(out + b_ref[...], 0.0)


def _tc_fused(x, neigh_x, w, bias2):
    grid = _R // _B
    return pl.pallas_call(
        _fused_body,
        grid=(grid,),
        in_specs=[
            pl.BlockSpec((_B, _D), lambda i: (i, 0)),
            pl.BlockSpec((_B, _S, _D), lambda i: (i, 0, 0)),
            pl.BlockSpec((_D, _U), lambda i: (0, 0)),
            pl.BlockSpec((1, _U), lambda i: (0, 0)),
        ],
        out_specs=pl.BlockSpec((_B, _U), lambda i: (i, 0)),
        out_shape=jax.ShapeDtypeStruct((_R, _U), jnp.float32),
        compiler_params=pltpu.CompilerParams(
            dimension_semantics=("arbitrary",),
        ),
    )(x, neigh_x, w, bias2)


def _proj_body(h_ref, x_ref, w_ref, b_ref, o_ref):
    h = (h_ref[...] + x_ref[...]) * _INV
    out = jnp.dot(h, w_ref[...], preferred_element_type=jnp.float32)
    o_ref[...] = jnp.maximum(out + b_ref[...], 0.0)


def _tc_project(h, x_tail, w, bias2):
    grid = _S_SC // _BP
    return pl.pallas_call(
        _proj_body,
        grid=(grid,),
        in_specs=[
            pl.BlockSpec((_BP, _D), lambda i: (i, 0)),
            pl.BlockSpec((_BP, _D), lambda i: (i, 0)),
            pl.BlockSpec((_D, _U), lambda i: (0, 0)),
            pl.BlockSpec((1, _U), lambda i: (0, 0)),
        ],
        out_specs=pl.BlockSpec((_BP, _U), lambda i: (i, 0)),
        out_shape=jax.ShapeDtypeStruct((_S_SC, _U), jnp.float32),
        compiler_params=pltpu.CompilerParams(
            dimension_semantics=("arbitrary",),
        ),
    )(h, x_tail, w, bias2)


def kernel(x, neigh_x, kernel, bias):
    bias2 = bias.reshape(1, _U)
    h_sc = _sc_aggregate(x, neigh_x)
    out_head = _tc_fused(x, neigh_x, kernel, bias2)
    out_tail = _tc_project(h_sc, lax.slice(x, (_R, 0), (_N, _D)), kernel,
                           bias2)
    return jnp.concatenate([out_head, out_tail], axis=0)
